# baseline (speedup 1.0000x reference)
import functools

import jax
import jax.numpy as jnp
from jax import lax
from jax.experimental import pallas as pl
from jax.experimental.pallas import tpu as pltpu
from jax.experimental.pallas import tpu_sc as plsc

B = 16
T = 2048
H = 1024
M = 64
K = 5
VOCAB = 64
KP = 8
NC = 32
IW = 128
TS = 16
TL = T // TS
ZROWS = 32


def _gate_topk_body(x_ref, wg_ref, wgc_ref, bg_ref, probs_ref, idx_ref):
    b = pl.program_id(0)
    x = x_ref[0]
    s = jnp.sum(x * wg_ref[...][None], axis=2) + bg_ref[0, 0]
    p = 1.0 / (1.0 + jnp.exp(-s))
    probs_ref[0] = p

    fi = (lax.broadcasted_iota(jnp.int32, (TS, TL), 0) * TL
          + lax.broadcasted_iota(jnp.int32, (TS, TL), 1))
    pw = s
    cands = []
    for _ in range(NC):
        mx = jnp.max(pw)
        cand = jnp.where(pw == mx, fi, T)
        am = jnp.min(cand)
        cands.append(am)
        pw = jnp.where(fi == am, -3e38, pw)

    rows = [x_ref[0, c // TL, pl.ds(c % TL, 1), :] for c in cands]
    xc = jnp.concatenate(rows, axis=0).astype(jnp.bfloat16)
    sc = jnp.dot(xc, wgc_ref[...],
                 preferred_element_type=jnp.float32)
    ci = lax.broadcasted_iota(jnp.int32, (NC, 1), 0)
    ctok = jnp.full((NC, 1), 0, jnp.int32)
    for j, c in enumerate(cands):
        ctok = jnp.where(ci == j, c, ctok)

    chosen = []
    for _ in range(K):
        mxs = jnp.max(sc)
        ctk = jnp.where(sc == mxs, ctok, T)
        tok = jnp.min(ctk)
        chosen.append(tok)
        sc = jnp.where(ctok == tok, -3e38, sc)

    base = b * T
    li = lax.broadcasted_iota(jnp.int32, (1, IW), 1)
    v = jnp.full((1, IW), base, jnp.int32)
    for j in range(K):
        v = jnp.where(li == j, base + chosen[j], v)
    idx_ref[0] = v


def _gate_topk(enc4, Wg, bg11):
    wg_row = Wg.reshape(1, H)
    wg_col16 = Wg.astype(jnp.bfloat16)
    return pl.pallas_call(
        _gate_topk_body,
        grid=(B,),
        in_specs=[
            pl.BlockSpec((1, TS, TL, H), lambda i: (i, 0, 0, 0)),
            pl.BlockSpec((1, H), lambda i: (0, 0)),
            pl.BlockSpec((H, 1), lambda i: (0, 0)),
            pl.BlockSpec((1, 1), lambda i: (0, 0)),
        ],
        out_specs=(
            pl.BlockSpec((1, TS, TL), lambda i: (i, 0, 0)),
            pl.BlockSpec((1, 1, IW), lambda i: (i, 0, 0)),
        ),
        out_shape=(
            jax.ShapeDtypeStruct((B, TS, TL), jnp.float32),
            jax.ShapeDtypeStruct((B, 1, IW), jnp.int32),
        ),
    )(enc4, wg_row, wg_col16, bg11)


def _sc_gather_memory(enc_flat, idx_flat, zrows):
    mesh = plsc.VectorSubcoreMesh(core_axis_name="c", subcore_axis_name="s")

    @functools.partial(
        pl.kernel,
        out_type=(
            jax.ShapeDtypeStruct((B * KP, H), jnp.float32),
            jax.ShapeDtypeStruct((B * M, H), jnp.float32),
        ),
        mesh=mesh,
        compiler_params=pltpu.CompilerParams(needs_layout_passes=False),
        scratch_types=[
            pltpu.VMEM((KP,), jnp.int32),
            pltpu.VMEM((KP, H), jnp.float32),
            pltpu.VMEM((ZROWS, H), jnp.float32),
            pltpu.SemaphoreType.DMA,
        ],
    )
    def body(enc_hbm, idx_hbm, z_hbm, g_hbm, mem_hbm, idx_v, rows_v, z_v,
             sem):
        w = lax.axis_index("s") * 2 + lax.axis_index("c")
        bb = lax.rem(w, B)
        half = w // B

        @pl.when(half == 0)
        def _():
            pltpu.sync_copy(idx_hbm.at[pl.ds(bb * IW, KP)], idx_v)
            pltpu.async_copy(enc_hbm.at[idx_v], rows_v, sem).wait()

            def zrow(c, carry):
                zv16 = jnp.zeros((16,), jnp.float32)
                for r in range(K, KP):
                    rows_v[r, pl.ds(c * 16, 16)] = zv16
                return carry

            lax.fori_loop(0, H // 16, zrow, 0)
            pltpu.sync_copy(rows_v, g_hbm.at[pl.ds(bb * KP, KP)])
            pltpu.sync_copy(rows_v, mem_hbm.at[pl.ds(bb * M, KP)])
            pltpu.sync_copy(z_hbm, z_v)
            pltpu.sync_copy(z_v, mem_hbm.at[pl.ds(bb * M + KP, ZROWS)])

        @pl.when(half == 1)
        def _():
            nz = M - KP - ZROWS
            pltpu.sync_copy(z_hbm.at[pl.ds(0, nz)], z_v.at[pl.ds(0, nz)])
            pltpu.sync_copy(z_v.at[pl.ds(0, nz)],
                            mem_hbm.at[pl.ds(bb * M + KP + ZROWS, nz)])

    return body(enc_flat, idx_flat, zrows)


def _read_body(g_ref, query_ref, wq_ref, bq_ref, wk_ref, bk_ref, wo_ref,
               bo_ref, logits_ref):
    slot = lax.broadcasted_iota(jnp.int32, (B, KP, H), 1)
    g = jnp.where(slot < K, g_ref[...].reshape(B, KP, H), 0.0)
    query = query_ref[...]
    q = jnp.dot(query, wq_ref[...],
                preferred_element_type=jnp.float32) + bq_ref[...]
    km = jnp.dot(g.reshape(B * KP, H), wk_ref[...],
                 preferred_element_type=jnp.float32).reshape(B, KP, H)
    km = km + bk_ref[...][None]
    inv = 1.0 / (H ** 0.5)
    z = jnp.sum(q * bk_ref[...], axis=1, keepdims=True) * inv
    s = jnp.sum(q[:, None, :] * km, axis=2) * inv
    mx = jnp.max(s, axis=1, keepdims=True)
    e = jnp.exp(s - mx)
    den = jnp.sum(e, axis=1, keepdims=True) + (M - KP) * jnp.exp(z - mx)
    attn = e / den
    retrieved = jnp.sum(attn[:, :, None] * g, axis=1)
    logits_ref[...] = jnp.dot(retrieved + query, wo_ref[...],
                              preferred_element_type=jnp.float32) + bo_ref[...]


def _read_path(g2, query_hidden, Wq, bq_row, Wk, bk_row, Wo, bo_row):
    return pl.pallas_call(
        _read_body,
        out_shape=jax.ShapeDtypeStruct((B, VOCAB), jnp.float32),
    )(g2, query_hidden, Wq, bq_row, Wk, bk_row, Wo, bo_row)


def kernel(enc_hidden, query_hidden, Wg, bg, Wq, bq, Wk, bk, Wo, bo):
    enc4 = enc_hidden.reshape(B, TS, TL, H)
    probs3, idx3 = _gate_topk(enc4, Wg, bg.reshape(1, 1))
    gate_probs = probs3.reshape(B, T)
    zrows = jnp.zeros((ZROWS, H), jnp.float32)
    gathered, mem_flat = _sc_gather_memory(
        enc_hidden.reshape(B * T, H), idx3.reshape(B * IW), zrows)
    logits = _read_path(
        gathered, query_hidden, Wq, bq.reshape(1, H),
        Wk, bk.reshape(1, H), Wo, bo.reshape(1, VOCAB))
    return (logits, gate_probs, mem_flat.reshape(B, M, H))


# speedup vs baseline: 2.6659x; 2.6659x over previous
import functools

import jax
import jax.numpy as jnp
from jax import lax
from jax.experimental import pallas as pl
from jax.experimental.pallas import tpu as pltpu
from jax.experimental.pallas import tpu_sc as plsc

B = 16
T = 2048
H = 1024
M = 64
K = 5
VOCAB = 64
KP = 8
NC = 32
IW = 128
TS = 16
TL = T // TS
ZROWS = 32


def _gate_topk_body(x_ref, wgc_ref, bg_ref, probs_ref, idx_ref):
    b = pl.program_id(0)
    x = x_ref[0].reshape(T, H).astype(jnp.bfloat16)
    s2 = jnp.dot(x, wgc_ref[...],
                 preferred_element_type=jnp.float32)
    s = s2.reshape(TS, TL) + bg_ref[0, 0]
    p = 1.0 / (1.0 + jnp.exp(-s))
    probs_ref[0] = p

    fi = (lax.broadcasted_iota(jnp.int32, (TS, TL), 0) * TL
          + lax.broadcasted_iota(jnp.int32, (TS, TL), 1))
    pw = s
    chosen = []
    for _ in range(K):
        mx = jnp.max(pw)
        cand = jnp.where(pw == mx, fi, T)
        am = jnp.min(cand)
        chosen.append(am)
        pw = jnp.where(fi == am, -3e38, pw)

    base = b * T
    li = lax.broadcasted_iota(jnp.int32, (1, IW), 1)
    v = jnp.full((1, IW), base, jnp.int32)
    for j in range(K):
        v = jnp.where(li == j, base + chosen[j], v)
    idx_ref[0] = v


def _gate_topk(enc4, Wg, bg11):
    wg_col16 = Wg.astype(jnp.bfloat16)
    return pl.pallas_call(
        _gate_topk_body,
        grid=(B,),
        in_specs=[
            pl.BlockSpec((1, TS, TL, H), lambda i: (i, 0, 0, 0)),
            pl.BlockSpec((H, 1), lambda i: (0, 0)),
            pl.BlockSpec((1, 1), lambda i: (0, 0)),
        ],
        out_specs=(
            pl.BlockSpec((1, TS, TL), lambda i: (i, 0, 0)),
            pl.BlockSpec((1, 1, IW), lambda i: (i, 0, 0)),
        ),
        out_shape=(
            jax.ShapeDtypeStruct((B, TS, TL), jnp.float32),
            jax.ShapeDtypeStruct((B, 1, IW), jnp.int32),
        ),
    )(enc4, wg_col16, bg11)


def _sc_gather_memory(enc_flat, idx_flat, zrows):
    mesh = plsc.VectorSubcoreMesh(core_axis_name="c", subcore_axis_name="s")

    @functools.partial(
        pl.kernel,
        out_type=(
            jax.ShapeDtypeStruct((B * KP, H), jnp.float32),
            jax.ShapeDtypeStruct((B * M, H), jnp.float32),
        ),
        mesh=mesh,
        compiler_params=pltpu.CompilerParams(needs_layout_passes=False),
        scratch_types=[
            pltpu.VMEM((KP,), jnp.int32),
            pltpu.VMEM((KP, H), jnp.float32),
            pltpu.VMEM((ZROWS, H), jnp.float32),
            pltpu.SemaphoreType.DMA,
        ],
    )
    def body(enc_hbm, idx_hbm, z_hbm, g_hbm, mem_hbm, idx_v, rows_v, z_v,
             sem):
        w = lax.axis_index("s") * 2 + lax.axis_index("c")
        bb = lax.rem(w, B)
        half = w // B

        @pl.when(half == 0)
        def _():
            pltpu.sync_copy(idx_hbm.at[pl.ds(bb * IW, KP)], idx_v)
            pltpu.async_copy(enc_hbm.at[idx_v], rows_v, sem).wait()

            def zrow(c, carry):
                zv16 = jnp.zeros((16,), jnp.float32)
                for r in range(K, KP):
                    rows_v[r, pl.ds(c * 16, 16)] = zv16
                return carry

            lax.fori_loop(0, H // 16, zrow, 0)
            pltpu.sync_copy(rows_v, g_hbm.at[pl.ds(bb * KP, KP)])
            pltpu.sync_copy(rows_v, mem_hbm.at[pl.ds(bb * M, KP)])
            pltpu.sync_copy(z_hbm, z_v)
            pltpu.sync_copy(z_v, mem_hbm.at[pl.ds(bb * M + KP, ZROWS)])

        @pl.when(half == 1)
        def _():
            nz = M - KP - ZROWS
            pltpu.sync_copy(z_hbm.at[pl.ds(0, nz)], z_v.at[pl.ds(0, nz)])
            pltpu.sync_copy(z_v.at[pl.ds(0, nz)],
                            mem_hbm.at[pl.ds(bb * M + KP + ZROWS, nz)])

    return body(enc_flat, idx_flat, zrows)


def _read_body(g_ref, query_ref, wq_ref, bq_ref, wk_ref, bk_ref, wo_ref,
               bo_ref, logits_ref):
    slot = lax.broadcasted_iota(jnp.int32, (B, KP, H), 1)
    g = jnp.where(slot < K, g_ref[...].reshape(B, KP, H), 0.0)
    query = query_ref[...]
    q = jnp.dot(query, wq_ref[...],
                preferred_element_type=jnp.float32) + bq_ref[...]
    km = jnp.dot(g.reshape(B * KP, H), wk_ref[...],
                 preferred_element_type=jnp.float32).reshape(B, KP, H)
    km = km + bk_ref[...][None]
    inv = 1.0 / (H ** 0.5)
    z = jnp.sum(q * bk_ref[...], axis=1, keepdims=True) * inv
    s = jnp.sum(q[:, None, :] * km, axis=2) * inv
    mx = jnp.max(s, axis=1, keepdims=True)
    e = jnp.exp(s - mx)
    den = jnp.sum(e, axis=1, keepdims=True) + (M - KP) * jnp.exp(z - mx)
    attn = e / den
    retrieved = jnp.sum(attn[:, :, None] * g, axis=1)
    logits_ref[...] = jnp.dot(retrieved + query, wo_ref[...],
                              preferred_element_type=jnp.float32) + bo_ref[...]


def _read_path(g2, query_hidden, Wq, bq_row, Wk, bk_row, Wo, bo_row):
    return pl.pallas_call(
        _read_body,
        out_shape=jax.ShapeDtypeStruct((B, VOCAB), jnp.float32),
    )(g2, query_hidden, Wq, bq_row, Wk, bk_row, Wo, bo_row)


def kernel(enc_hidden, query_hidden, Wg, bg, Wq, bq, Wk, bk, Wo, bo):
    enc4 = enc_hidden.reshape(B, TS, TL, H)
    probs3, idx3 = _gate_topk(enc4, Wg, bg.reshape(1, 1))
    gate_probs = probs3.reshape(B, T)
    zrows = jnp.zeros((ZROWS, H), jnp.float32)
    gathered, mem_flat = _sc_gather_memory(
        enc_hidden.reshape(B * T, H), idx3.reshape(B * IW), zrows)
    logits = _read_path(
        gathered, query_hidden, Wq, bq.reshape(1, H),
        Wk, bk.reshape(1, H), Wo, bo.reshape(1, VOCAB))
    return (logits, gate_probs, mem_flat.reshape(B, M, H))


# speedup vs baseline: 2.7249x; 1.0222x over previous
import functools

import jax
import jax.numpy as jnp
from jax import lax
from jax.experimental import pallas as pl
from jax.experimental.pallas import tpu as pltpu
from jax.experimental.pallas import tpu_sc as plsc

B = 16
T = 2048
H = 1024
M = 64
K = 5
VOCAB = 64
KP = 8
NC = 32
IW = 128
TS = 16
TL = T // TS
ZROWS = 32


def _gate_topk_body(x_ref, wgc_ref, bg_ref, probs_ref, idx_ref, g_ref):
    b = pl.program_id(0)
    x = x_ref[0].reshape(T, H).astype(jnp.bfloat16)
    s2 = jnp.dot(x, wgc_ref[...],
                 preferred_element_type=jnp.float32)
    s = s2.reshape(TS, TL) + bg_ref[0, 0]
    p = 1.0 / (1.0 + jnp.exp(-s))
    probs_ref[0] = p

    fi = (lax.broadcasted_iota(jnp.int32, (TS, TL), 0) * TL
          + lax.broadcasted_iota(jnp.int32, (TS, TL), 1))
    pw = s
    chosen = []
    for _ in range(K):
        mx = jnp.max(pw)
        cand = jnp.where(pw == mx, fi, T)
        am = jnp.min(cand)
        chosen.append(am)
        pw = jnp.where(fi == am, -3e38, pw)

    base = b * T
    li = lax.broadcasted_iota(jnp.int32, (1, IW), 1)
    v = jnp.full((1, IW), base, jnp.int32)
    for j in range(K):
        v = jnp.where(li == j, base + chosen[j], v)
    idx_ref[0] = v

    rows = [x_ref[0, c // TL, pl.ds(lax.rem(c, TL), 1), :] for c in chosen]
    g_ref[0] = jnp.concatenate(
        rows + [jnp.zeros((KP - K, H), jnp.float32)], axis=0)


def _gate_topk(enc4, Wg, bg11):
    wg_col16 = Wg.astype(jnp.bfloat16)
    return pl.pallas_call(
        _gate_topk_body,
        grid=(B,),
        in_specs=[
            pl.BlockSpec((1, TS, TL, H), lambda i: (i, 0, 0, 0)),
            pl.BlockSpec((H, 1), lambda i: (0, 0)),
            pl.BlockSpec((1, 1), lambda i: (0, 0)),
        ],
        out_specs=(
            pl.BlockSpec((1, TS, TL), lambda i: (i, 0, 0)),
            pl.BlockSpec((1, 1, IW), lambda i: (i, 0, 0)),
            pl.BlockSpec((1, KP, H), lambda i: (i, 0, 0)),
        ),
        out_shape=(
            jax.ShapeDtypeStruct((B, TS, TL), jnp.float32),
            jax.ShapeDtypeStruct((B, 1, IW), jnp.int32),
            jax.ShapeDtypeStruct((B, KP, H), jnp.float32),
        ),
    )(enc4, wg_col16, bg11)


def _sc_gather_memory(enc_flat, idx_flat, zrows):
    mesh = plsc.VectorSubcoreMesh(core_axis_name="c", subcore_axis_name="s")

    @functools.partial(
        pl.kernel,
        out_type=jax.ShapeDtypeStruct((B * M, H), jnp.float32),
        mesh=mesh,
        compiler_params=pltpu.CompilerParams(needs_layout_passes=False),
        scratch_types=[
            pltpu.VMEM((KP,), jnp.int32),
            pltpu.VMEM((KP, H), jnp.float32),
            pltpu.VMEM((ZROWS, H), jnp.float32),
            pltpu.SemaphoreType.DMA,
        ],
    )
    def body(enc_hbm, idx_hbm, z_hbm, mem_hbm, idx_v, rows_v, z_v, sem):
        w = lax.axis_index("s") * 2 + lax.axis_index("c")
        bb = lax.rem(w, B)
        half = w // B

        @pl.when(half == 0)
        def _():
            pltpu.sync_copy(idx_hbm.at[pl.ds(bb * IW, KP)], idx_v)
            pltpu.async_copy(enc_hbm.at[idx_v], rows_v, sem).wait()

            def zrow(c, carry):
                zv16 = jnp.zeros((16,), jnp.float32)
                for r in range(K, KP):
                    rows_v[r, pl.ds(c * 16, 16)] = zv16
                return carry

            lax.fori_loop(0, H // 16, zrow, 0)
            pltpu.sync_copy(rows_v, mem_hbm.at[pl.ds(bb * M, KP)])
            pltpu.sync_copy(z_hbm, z_v)
            pltpu.sync_copy(z_v, mem_hbm.at[pl.ds(bb * M + KP, ZROWS)])

        @pl.when(half == 1)
        def _():
            nz = M - KP - ZROWS
            pltpu.sync_copy(z_hbm.at[pl.ds(0, nz)], z_v.at[pl.ds(0, nz)])
            pltpu.sync_copy(z_v.at[pl.ds(0, nz)],
                            mem_hbm.at[pl.ds(bb * M + KP + ZROWS, nz)])

    return body(enc_flat, idx_flat, zrows)


def _read_body(g_ref, query_ref, wq_ref, bq_ref, wk_ref, bk_ref, wo_ref,
               bo_ref, logits_ref):
    slot = lax.broadcasted_iota(jnp.int32, (B, KP, H), 1)
    g = jnp.where(slot < K, g_ref[...].reshape(B, KP, H), 0.0)
    query = query_ref[...]
    q = jnp.dot(query, wq_ref[...],
                preferred_element_type=jnp.float32) + bq_ref[...]
    km = jnp.dot(g.reshape(B * KP, H), wk_ref[...],
                 preferred_element_type=jnp.float32).reshape(B, KP, H)
    km = km + bk_ref[...][None]
    inv = 1.0 / (H ** 0.5)
    z = jnp.sum(q * bk_ref[...], axis=1, keepdims=True) * inv
    s = jnp.sum(q[:, None, :] * km, axis=2) * inv
    mx = jnp.max(s, axis=1, keepdims=True)
    e = jnp.exp(s - mx)
    den = jnp.sum(e, axis=1, keepdims=True) + (M - KP) * jnp.exp(z - mx)
    attn = e / den
    retrieved = jnp.sum(attn[:, :, None] * g, axis=1)
    logits_ref[...] = jnp.dot(retrieved + query, wo_ref[...],
                              preferred_element_type=jnp.float32) + bo_ref[...]


def _read_path(g2, query_hidden, Wq, bq_row, Wk, bk_row, Wo, bo_row):
    return pl.pallas_call(
        _read_body,
        out_shape=jax.ShapeDtypeStruct((B, VOCAB), jnp.float32),
    )(g2, query_hidden, Wq, bq_row, Wk, bk_row, Wo, bo_row)


def kernel(enc_hidden, query_hidden, Wg, bg, Wq, bq, Wk, bk, Wo, bo):
    enc4 = enc_hidden.reshape(B, TS, TL, H)
    probs3, idx3, g3 = _gate_topk(enc4, Wg, bg.reshape(1, 1))
    gate_probs = probs3.reshape(B, T)
    zrows = jnp.zeros((ZROWS, H), jnp.float32)
    mem_flat = _sc_gather_memory(
        enc_hidden.reshape(B * T, H), idx3.reshape(B * IW), zrows)
    logits = _read_path(
        g3.reshape(B * KP, H), query_hidden, Wq, bq.reshape(1, H),
        Wk, bk.reshape(1, H), Wo, bo.reshape(1, VOCAB))
    return (logits, gate_probs, mem_flat.reshape(B, M, H))
